# baseline (speedup 1.0000x reference)
import dataclasses

import jax
import jax.numpy as jnp
from jax import lax
from jax.experimental import pallas as pl
from jax.experimental.pallas import tpu as pltpu
from jax.experimental.pallas import tpu_sc as plsc

N = 10000
E = 320000
D_IN = 128
H = 64
G = 64

NPAD = 10240
WIDTH = 128
SCALED = 80
NTILES = 32
C = 96
ETOT = E + N
K = ((ETOT + NTILES * C - 1) // (NTILES * C) + 1) // 2 * 2
K_A = 92
K_B = 2 * K - K_A
EPAD = 16 * (K_A + K_B) * C

_f32 = jnp.float32


def _scores_from_h(h, a_ref, hpad_ref, sc_ref):
    hpad_ref[:, 0:64] = h
    lane = lax.broadcasted_iota(jnp.int32, (NPAD, 64), 1)
    hpad_ref[:, 64:128] = jnp.where(lane == 0, 1.0, 0.0).astype(_f32)
    as2 = jnp.dot(h, a_ref[...], preferred_element_type=_f32)
    sc_ref[:, 0:2] = as2
    m = jnp.max(as2[:, 0:1]) + jnp.max(as2[:, 1:2])
    m = jnp.maximum(m, 0.2 * m)
    sc_ref[:, 2:8] = jnp.full((NPAD, 6), m, dtype=_f32)


def _prep1_body(x_ref, w_ref, a_ref, hpad_ref, sc_ref):
    h = jnp.dot(x_ref[...], w_ref[...], preferred_element_type=_f32)
    _scores_from_h(h, a_ref, hpad_ref, sc_ref)


def _prep_mid_body(acc_ref, b_ref, w_ref, a_ref, hpad_ref, sc_ref):
    a = acc_ref[0] + acc_ref[1]
    num = a[:, 0:64]
    den = a[:, 64:65]
    hprev = jnp.maximum(num / (den + 1e-16) + b_ref[...], 0.0)
    row = lax.broadcasted_iota(jnp.int32, (NPAD, 64), 0)
    hprev = jnp.where(row < N, hprev, 0.0)
    h = jnp.dot(hprev, w_ref[...], preferred_element_type=_f32)
    _scores_from_h(h, a_ref, hpad_ref, sc_ref)


def _final_body(acc_ref, b_ref, batch_ref, wl_ref, bl_ref, out_ref):
    a = acc_ref[0] + acc_ref[1]
    num = a[:, 0:64]
    den = a[:, 64:65]
    h3 = num / (den + 1e-16) + b_ref[...]
    h3 = lax.slice(h3, (0, 0), (N, 64))
    bt = batch_ref[0:1, :]
    gidx = lax.broadcasted_iota(jnp.int32, (G, N), 0)
    oh = jnp.where(bt == gidx, 1.0, 0.0).astype(_f32)
    sums = jnp.dot(oh, h3, preferred_element_type=_f32)
    cnt = jnp.dot(oh, jnp.ones((N, 1), _f32),
                  preferred_element_type=_f32)
    pooled = sums / jnp.maximum(cnt, 1.0)
    out_ref[...] = jnp.dot(pooled, wl_ref[...],
                           preferred_element_type=_f32) + bl_ref[...]


def _tc_prep1(x_pad, W1, a2d):
    return pl.pallas_call(
        _prep1_body,
        out_shape=[jax.ShapeDtypeStruct((NPAD, WIDTH), _f32),
                   jax.ShapeDtypeStruct((NPAD, 8), _f32)],
    )(x_pad, W1, a2d)


def _tc_prep_mid(acc, b2d, W, a2d):
    return pl.pallas_call(
        _prep_mid_body,
        out_shape=[jax.ShapeDtypeStruct((NPAD, WIDTH), _f32),
                   jax.ShapeDtypeStruct((NPAD, 8), _f32)],
    )(acc, b2d, W, a2d)


def _tc_final(acc, b2d, batchb, Wl, bl2d):
    return pl.pallas_call(
        _final_body,
        out_shape=jax.ShapeDtypeStruct((G, H), _f32),
    )(acc, b2d, batchb, Wl, bl2d)


def _sc_body(hpad_hbm, asv_hbm, adv_hbm, mvec_hbm, src_hbm, dst_hbm,
             zeros_hbm, out_hbm,
             as_t, ad_t, mv,
             srcv0, dstv0, srcv1, dstv1, dsc0, dsc1, rows0, rows1, exv,
             sidx0, sidx1, srow0, srow1, ssc0, ssc1, acc_sh):
    cid = lax.axis_index("c")
    sid = lax.axis_index("s")
    wid = sid * 2 + cid

    pltpu.sync_copy(asv_hbm, as_t)
    pltpu.sync_copy(adv_hbm, ad_t)
    pltpu.sync_copy(mvec_hbm, mv)

    @pl.when(sid == 0)
    def _():
        pltpu.sync_copy(zeros_hbm, acc_sh)

    plsc.subcore_barrier()

    mvv = mv[...]
    base0 = jnp.where(cid == 0, sid * (K_A * C),
                      16 * (K_A * C) + sid * (K_B * C))
    myK2 = jnp.where(cid == 0, K_A // 2, K_B // 2)

    def start_idx(k, sv, dv, sem):
        base = pl.multiple_of(base0 + k * C, C)
        pltpu.async_copy(src_hbm.at[pl.ds(base, C)], sv, sem)
        pltpu.async_copy(dst_hbm.at[pl.ds(base, C)], dv, sem)

    def wait_idx(k, sv, dv, sem):
        base = pl.multiple_of(base0 + k * C, C)
        pltpu.make_async_copy(src_hbm.at[pl.ds(base, C)], sv, sem).wait()
        pltpu.make_async_copy(dst_hbm.at[pl.ds(base, C)], dv, sem).wait()

    def scores(sv, dv):
        for g in range(C // 16):
            s16 = sv[pl.ds(g * 16, 16)]
            d16 = dv[pl.ds(g * 16, 16)]
            e = plsc.load_gather(as_t, [s16]) + plsc.load_gather(ad_t, [d16])
            e = jnp.maximum(e, 0.2 * e)
            exv[pl.ds(g * 16, 16)] = jnp.exp(e - mvv)

    def copy_dst(dv, dsc):
        for g in range(C // 16):
            dsc[pl.ds(g * 16, 16)] = dv[pl.ds(g * 16, 16)]

    def scale(rows):
        @plsc.parallel_loop(0, C, 1, unroll=8)
        def _row(j):
            jj = jnp.full((16,), j, jnp.int32)
            sp = plsc.load_gather(exv, [jj])
            r = rows.at[j]
            for q in range(SCALED // 16):
                r[pl.ds(q * 16, 16)] = r[pl.ds(q * 16, 16)] * sp

    def half(i, k, sv, dv, dsc, rows, my_sidx, my_srow, my_ssc,
             ot_sv, ot_dv, ot_dsc, ot_rows, ot_sidx, ot_srow, ot_ssc,
             scat_wait_cond, gather_cond, K2):
        scores(sv, dv)
        copy_dst(dv, dsc)
        pltpu.make_async_copy(hpad_hbm.at[sv], rows, my_srow).wait()

        @pl.when(i < K2 - 1)
        def _():
            start_idx(k + 2, sv, dv, my_sidx)

        scale(rows)

        @pl.when(scat_wait_cond)
        def _():
            pltpu.make_async_copy(ot_rows, acc_sh.at[ot_dsc], ot_ssc).wait()

        @pl.when(gather_cond)
        def _():
            wait_idx(k + 1, ot_sv, ot_dv, ot_sidx)
            pltpu.async_copy(hpad_hbm.at[ot_sv], ot_rows, ot_srow)

        pltpu.async_copy(rows, acc_sh.at[dsc], my_ssc, add=True)

    start_idx(0, srcv0, dstv0, sidx0)
    start_idx(1, srcv1, dstv1, sidx1)
    wait_idx(0, srcv0, dstv0, sidx0)
    pltpu.async_copy(hpad_hbm.at[srcv0], rows0, srow0)

    @pl.loop(0, myK2)
    def _it(i):
        k0 = 2 * i
        half(i, k0, srcv0, dstv0, dsc0, rows0, sidx0, srow0, ssc0,
             srcv1, dstv1, dsc1, rows1, sidx1, srow1, ssc1,
             i > 0, i >= 0, myK2)
        half(i, k0 + 1, srcv1, dstv1, dsc1, rows1, sidx1, srow1, ssc1,
             srcv0, dstv0, dsc0, rows0, sidx0, srow0, ssc0,
             i < myK2 - 1, i < myK2 - 1, myK2)

    pltpu.make_async_copy(rows0, acc_sh.at[dsc0], ssc0).wait()
    pltpu.make_async_copy(rows1, acc_sh.at[dsc1], ssc1).wait()

    plsc.subcore_barrier()
    rpt = NPAD // 16
    pltpu.sync_copy(acc_sh.at[pl.ds(sid * rpt, rpt)],
                    out_hbm.at[cid, pl.ds(sid * rpt, rpt)])


def _sc_compiler_params():
    cp = pltpu.CompilerParams()
    if "needs_layout_passes" in pltpu.CompilerParams.__dataclass_fields__:
        cp = dataclasses.replace(cp, needs_layout_passes=False)
    return cp


def _sc_layer(hpad, asv, adv, mvec, srcp, dstp, zeros):
    mesh = plsc.VectorSubcoreMesh(core_axis_name="c", subcore_axis_name="s")
    f = pl.kernel(
        _sc_body,
        out_type=jax.ShapeDtypeStruct((2, NPAD, WIDTH), _f32),
        mesh=mesh,
        compiler_params=_sc_compiler_params(),
        scratch_types=[
            pltpu.VMEM((NPAD,), _f32),
            pltpu.VMEM((NPAD,), _f32),
            pltpu.VMEM((16,), _f32),
            pltpu.VMEM((C,), jnp.int32),
            pltpu.VMEM((C,), jnp.int32),
            pltpu.VMEM((C,), jnp.int32),
            pltpu.VMEM((C,), jnp.int32),
            pltpu.VMEM((C,), jnp.int32),
            pltpu.VMEM((C,), jnp.int32),
            pltpu.VMEM((C, WIDTH), _f32),
            pltpu.VMEM((C, WIDTH), _f32),
            pltpu.VMEM((C,), _f32),
            pltpu.SemaphoreType.DMA,
            pltpu.SemaphoreType.DMA,
            pltpu.SemaphoreType.DMA,
            pltpu.SemaphoreType.DMA,
            pltpu.SemaphoreType.DMA,
            pltpu.SemaphoreType.DMA,
            pltpu.VMEM_SHARED((NPAD, WIDTH), _f32),
        ],
    )
    return f(hpad, asv, adv, mvec, srcp, dstp, zeros)


@jax.jit
def kernel(x, edge_index, batch, W1, a_src1, a_dst1, b1, W2, a_src2, a_dst2,
           b2, W3, a_src3, a_dst3, b3, Wl, bl):
    idt = edge_index.dtype
    loop = jnp.arange(N, dtype=idt)
    srcp = jnp.concatenate(
        [edge_index[0], loop,
         jnp.zeros((EPAD - ETOT,), idt)]).astype(jnp.int32)
    dstp = jnp.concatenate(
        [edge_index[1], loop,
         jnp.full((EPAD - ETOT,), N, idt)]).astype(jnp.int32)
    x_pad = jnp.pad(x, ((0, NPAD - N), (0, 0)))
    zeros = jnp.zeros((NPAD, WIDTH), _f32)
    batchb = jnp.broadcast_to(batch.astype(jnp.int32)[None, :], (8, N))

    def layer(hpad, scores):
        asv = scores[:, 0]
        adv = scores[:, 1]
        mvec = scores[0:16, 2]
        return _sc_layer(hpad, asv, adv, mvec, srcp, dstp, zeros)

    a2d1 = jnp.stack([a_src1, a_dst1], axis=1)
    a2d2 = jnp.stack([a_src2, a_dst2], axis=1)
    a2d3 = jnp.stack([a_src3, a_dst3], axis=1)

    hpad, scores = _tc_prep1(x_pad, W1, a2d1)
    acc = layer(hpad, scores)
    hpad, scores = _tc_prep_mid(acc, b1[None, :], W2, a2d2)
    acc = layer(hpad, scores)
    hpad, scores = _tc_prep_mid(acc, b2[None, :], W3, a2d3)
    acc = layer(hpad, scores)
    return _tc_final(acc, b3[None, :], batchb, Wl, bl[None, :])


# speedup vs baseline: 1.1492x; 1.1492x over previous
import dataclasses

import jax
import jax.numpy as jnp
from jax import lax
from jax.experimental import pallas as pl
from jax.experimental.pallas import tpu as pltpu
from jax.experimental.pallas import tpu_sc as plsc

N = 10000
E = 320000
D_IN = 128
H = 64
G = 64

NPAD = 10240
WIDTH = 128
SCALED = 80
NTILES = 32
C = 96
ETOT = E + N
K = ((ETOT + NTILES * C - 1) // (NTILES * C) + 1) // 2 * 2
K_A = 124
K_B = 2 * K - K_A
EPAD = 16 * (K_A + K_B) * C

_f32 = jnp.float32


def _scores_from_h(h, a_ref, hpad_ref, sc_ref):
    hpad_ref[:, 0:64] = h
    lane = lax.broadcasted_iota(jnp.int32, (NPAD, 64), 1)
    hpad_ref[:, 64:128] = jnp.where(lane == 0, 1.0, 0.0).astype(_f32)
    as2 = jnp.dot(h, a_ref[...], preferred_element_type=_f32)
    sc_ref[:, 0:2] = as2
    m = jnp.max(as2[:, 0:1]) + jnp.max(as2[:, 1:2])
    m = jnp.maximum(m, 0.2 * m)
    sc_ref[:, 2:8] = jnp.full((NPAD, 6), m, dtype=_f32)


def _prep1_body(x_ref, w_ref, a_ref, hpad_ref, sc_ref):
    h = jnp.dot(x_ref[...], w_ref[...], preferred_element_type=_f32)
    _scores_from_h(h, a_ref, hpad_ref, sc_ref)


def _prep_mid_body(acc_ref, b_ref, w_ref, a_ref, hpad_ref, sc_ref):
    a = acc_ref[0] + acc_ref[1]
    num = a[:, 0:64]
    den = a[:, 64:65]
    hprev = jnp.maximum(num / (den + 1e-16) + b_ref[...], 0.0)
    row = lax.broadcasted_iota(jnp.int32, (NPAD, 64), 0)
    hprev = jnp.where(row < N, hprev, 0.0)
    h = jnp.dot(hprev, w_ref[...], preferred_element_type=_f32)
    _scores_from_h(h, a_ref, hpad_ref, sc_ref)


def _final_body(acc_ref, b_ref, batch_ref, wl_ref, bl_ref, out_ref):
    a = acc_ref[0] + acc_ref[1]
    num = a[:, 0:64]
    den = a[:, 64:65]
    h3 = num / (den + 1e-16) + b_ref[...]
    h3 = lax.slice(h3, (0, 0), (N, 64))
    bt = batch_ref[0:1, :]
    gidx = lax.broadcasted_iota(jnp.int32, (G, N), 0)
    oh = jnp.where(bt == gidx, 1.0, 0.0).astype(_f32)
    sums = jnp.dot(oh, h3, preferred_element_type=_f32)
    cnt = jnp.dot(oh, jnp.ones((N, 1), _f32),
                  preferred_element_type=_f32)
    pooled = sums / jnp.maximum(cnt, 1.0)
    out_ref[...] = jnp.dot(pooled, wl_ref[...],
                           preferred_element_type=_f32) + bl_ref[...]


def _tc_prep1(x_pad, W1, a2d):
    return pl.pallas_call(
        _prep1_body,
        out_shape=[jax.ShapeDtypeStruct((NPAD, WIDTH), _f32),
                   jax.ShapeDtypeStruct((NPAD, 8), _f32)],
    )(x_pad, W1, a2d)


def _tc_prep_mid(acc, b2d, W, a2d):
    return pl.pallas_call(
        _prep_mid_body,
        out_shape=[jax.ShapeDtypeStruct((NPAD, WIDTH), _f32),
                   jax.ShapeDtypeStruct((NPAD, 8), _f32)],
    )(acc, b2d, W, a2d)


def _tc_final(acc, b2d, batchb, Wl, bl2d):
    return pl.pallas_call(
        _final_body,
        out_shape=jax.ShapeDtypeStruct((G, H), _f32),
    )(acc, b2d, batchb, Wl, bl2d)


def _sc_body(hpad_hbm, asv_hbm, adv_hbm, mvec_hbm, src_hbm, dst_hbm,
             zeros_hbm, out_hbm,
             as_t, ad_t, mv,
             srcv0, dstv0, srcv1, dstv1, dsc0, dsc1, rows0, rows1, exv,
             sidx0, sidx1, srow0, srow1, ssc0, ssc1, acc_sh):
    cid = lax.axis_index("c")
    sid = lax.axis_index("s")
    wid = sid * 2 + cid

    pltpu.sync_copy(asv_hbm, as_t)
    pltpu.sync_copy(adv_hbm, ad_t)
    pltpu.sync_copy(mvec_hbm, mv)

    @pl.when(sid == 0)
    def _():
        pltpu.sync_copy(zeros_hbm, acc_sh)

    plsc.subcore_barrier()

    mvv = mv[...]
    base0 = jnp.where(cid == 0, sid * (K_A * C),
                      16 * (K_A * C) + sid * (K_B * C))
    myK2 = jnp.where(cid == 0, K_A // 2, K_B // 2)

    def start_idx(k, sv, dv, sem):
        base = pl.multiple_of(base0 + k * C, C)
        pltpu.async_copy(src_hbm.at[pl.ds(base, C)], sv, sem)
        pltpu.async_copy(dst_hbm.at[pl.ds(base, C)], dv, sem)

    def wait_idx(k, sv, dv, sem):
        base = pl.multiple_of(base0 + k * C, C)
        pltpu.make_async_copy(src_hbm.at[pl.ds(base, C)], sv, sem).wait()
        pltpu.make_async_copy(dst_hbm.at[pl.ds(base, C)], dv, sem).wait()

    def scores(sv, dv):
        for g in range(C // 16):
            s16 = sv[pl.ds(g * 16, 16)]
            d16 = dv[pl.ds(g * 16, 16)]
            e = plsc.load_gather(as_t, [s16]) + plsc.load_gather(ad_t, [d16])
            e = jnp.maximum(e, 0.2 * e)
            exv[pl.ds(g * 16, 16)] = jnp.exp(e - mvv)

    def copy_dst(dv, dsc):
        for g in range(C // 16):
            dsc[pl.ds(g * 16, 16)] = dv[pl.ds(g * 16, 16)]

    def scale(rows):
        @plsc.parallel_loop(0, C, 1, unroll=8)
        def _row(j):
            jj = jnp.full((16,), j, jnp.int32)
            sp = plsc.load_gather(exv, [jj])
            r = rows.at[j]
            for q in range(SCALED // 16):
                r[pl.ds(q * 16, 16)] = r[pl.ds(q * 16, 16)] * sp

    def half(i, k, sv, dv, dsc, rows, my_sidx, my_srow, my_ssc,
             ot_sv, ot_dv, ot_dsc, ot_rows, ot_sidx, ot_srow, ot_ssc,
             scat_wait_cond, gather_cond, K2):
        scores(sv, dv)
        copy_dst(dv, dsc)
        pltpu.make_async_copy(hpad_hbm.at[sv], rows, my_srow).wait()

        @pl.when(i < K2 - 1)
        def _():
            start_idx(k + 2, sv, dv, my_sidx)

        scale(rows)

        @pl.when(scat_wait_cond)
        def _():
            pltpu.make_async_copy(ot_rows, acc_sh.at[ot_dsc], ot_ssc).wait()

        @pl.when(gather_cond)
        def _():
            wait_idx(k + 1, ot_sv, ot_dv, ot_sidx)
            pltpu.async_copy(hpad_hbm.at[ot_sv], ot_rows, ot_srow)

        pltpu.async_copy(rows, acc_sh.at[dsc], my_ssc, add=True)

    start_idx(0, srcv0, dstv0, sidx0)
    start_idx(1, srcv1, dstv1, sidx1)
    wait_idx(0, srcv0, dstv0, sidx0)
    pltpu.async_copy(hpad_hbm.at[srcv0], rows0, srow0)

    @pl.loop(0, myK2)
    def _it(i):
        k0 = 2 * i
        half(i, k0, srcv0, dstv0, dsc0, rows0, sidx0, srow0, ssc0,
             srcv1, dstv1, dsc1, rows1, sidx1, srow1, ssc1,
             i > 0, i >= 0, myK2)
        half(i, k0 + 1, srcv1, dstv1, dsc1, rows1, sidx1, srow1, ssc1,
             srcv0, dstv0, dsc0, rows0, sidx0, srow0, ssc0,
             i < myK2 - 1, i < myK2 - 1, myK2)

    pltpu.make_async_copy(rows0, acc_sh.at[dsc0], ssc0).wait()
    pltpu.make_async_copy(rows1, acc_sh.at[dsc1], ssc1).wait()

    plsc.subcore_barrier()
    rpt = NPAD // 16
    pltpu.sync_copy(acc_sh.at[pl.ds(sid * rpt, rpt)],
                    out_hbm.at[cid, pl.ds(sid * rpt, rpt)])


def _sc_compiler_params():
    cp = pltpu.CompilerParams()
    if "needs_layout_passes" in pltpu.CompilerParams.__dataclass_fields__:
        cp = dataclasses.replace(cp, needs_layout_passes=False)
    return cp


def _sc_layer(hpad, asv, adv, mvec, srcp, dstp, zeros):
    mesh = plsc.VectorSubcoreMesh(core_axis_name="c", subcore_axis_name="s")
    f = pl.kernel(
        _sc_body,
        out_type=jax.ShapeDtypeStruct((2, NPAD, WIDTH), _f32),
        mesh=mesh,
        compiler_params=_sc_compiler_params(),
        scratch_types=[
            pltpu.VMEM((NPAD,), _f32),
            pltpu.VMEM((NPAD,), _f32),
            pltpu.VMEM((16,), _f32),
            pltpu.VMEM((C,), jnp.int32),
            pltpu.VMEM((C,), jnp.int32),
            pltpu.VMEM((C,), jnp.int32),
            pltpu.VMEM((C,), jnp.int32),
            pltpu.VMEM((C,), jnp.int32),
            pltpu.VMEM((C,), jnp.int32),
            pltpu.VMEM((C, WIDTH), _f32),
            pltpu.VMEM((C, WIDTH), _f32),
            pltpu.VMEM((C,), _f32),
            pltpu.SemaphoreType.DMA,
            pltpu.SemaphoreType.DMA,
            pltpu.SemaphoreType.DMA,
            pltpu.SemaphoreType.DMA,
            pltpu.SemaphoreType.DMA,
            pltpu.SemaphoreType.DMA,
            pltpu.VMEM_SHARED((NPAD, WIDTH), _f32),
        ],
    )
    return f(hpad, asv, adv, mvec, srcp, dstp, zeros)


@jax.jit
def kernel(x, edge_index, batch, W1, a_src1, a_dst1, b1, W2, a_src2, a_dst2,
           b2, W3, a_src3, a_dst3, b3, Wl, bl):
    idt = edge_index.dtype
    loop = jnp.arange(N, dtype=idt)
    srcp = jnp.concatenate(
        [edge_index[0], loop,
         jnp.zeros((EPAD - ETOT,), idt)]).astype(jnp.int32)
    dstp = jnp.concatenate(
        [edge_index[1], loop,
         jnp.full((EPAD - ETOT,), N, idt)]).astype(jnp.int32)
    x_pad = jnp.pad(x, ((0, NPAD - N), (0, 0)))
    zeros = jnp.zeros((NPAD, WIDTH), _f32)
    batchb = jnp.broadcast_to(batch.astype(jnp.int32)[None, :], (8, N))

    def layer(hpad, scores):
        asv = scores[:, 0]
        adv = scores[:, 1]
        mvec = scores[0:16, 2]
        return _sc_layer(hpad, asv, adv, mvec, srcp, dstp, zeros)

    a2d1 = jnp.stack([a_src1, a_dst1], axis=1)
    a2d2 = jnp.stack([a_src2, a_dst2], axis=1)
    a2d3 = jnp.stack([a_src3, a_dst3], axis=1)

    hpad, scores = _tc_prep1(x_pad, W1, a2d1)
    acc = layer(hpad, scores)
    hpad, scores = _tc_prep_mid(acc, b1[None, :], W2, a2d2)
    acc = layer(hpad, scores)
    hpad, scores = _tc_prep_mid(acc, b2[None, :], W3, a2d3)
    acc = layer(hpad, scores)
    return _tc_final(acc, b3[None, :], batchb, Wl, bl[None, :])
